# baseline (speedup 1.0000x reference)
import functools

import jax
import jax.numpy as jnp
from jax import lax
from jax.experimental import pallas as pl
from jax.experimental.pallas import tpu as pltpu
from jax.experimental.pallas import tpu_sc as plsc

NE = 8
D = 1024
F = 1024
BS = 4096
NA = 2 * BS
G = 256
NB = (NA + NE * (G - 1) + G - 1) // G
PAD = NB * G

_LANES = 128

NW = 32
SC_CH = 64


def _route_body(x_ref, wr_ref, inv0_ref, inv1_ref, w0_ref, w1_ref, be_ref,
                rank_ref):
    x = x_ref[...]
    wr = wr_ref[...]
    logits = jnp.dot(x, wr, preferred_element_type=jnp.float32)
    lane = lax.broadcasted_iota(jnp.int32, (BS, _LANES), 1)
    valid = lane < NE
    neg = jnp.float32(-1e30)
    ml = jnp.where(valid, logits, neg)

    m1 = jnp.max(ml, axis=1, keepdims=True)
    idx1 = jnp.min(jnp.where(ml == m1, lane, _LANES), axis=1, keepdims=True)
    oh0 = (lane == idx1).astype(jnp.float32)
    ml2 = jnp.where(lane == idx1, neg, ml)
    m2 = jnp.max(ml2, axis=1, keepdims=True)
    idx2 = jnp.min(jnp.where(ml2 == m2, lane, _LANES), axis=1, keepdims=True)
    oh1 = (lane == idx2).astype(jnp.float32)

    w0_ref[...] = jax.nn.sigmoid(m1 - m2)
    w1_ref[...] = jax.nn.sigmoid(m2 - m1)

    rank_ref[...] = oh0 + oh1
    tri = (lax.broadcasted_iota(jnp.int32, (128, 128), 0) >
           lax.broadcasted_iota(jnp.int32, (128, 128), 1)).astype(jnp.float32)

    def chunk(c, carry):
        ch = rank_ref[pl.ds(c * 128, 128), :]
        within = jnp.dot(tri, ch, preferred_element_type=jnp.float32)
        rank_ref[pl.ds(c * 128, 128), :] = within + carry
        return carry + jnp.sum(ch, axis=0, keepdims=True)

    counts = lax.fori_loop(0, BS // 128, chunk,
                           jnp.zeros((1, _LANES), jnp.float32))
    rank = rank_ref[...]

    pc = jnp.floor((counts + (G - 1)) * (1.0 / G)) * G
    upper = (lax.broadcasted_iota(jnp.int32, (128, 128), 0) <
             lax.broadcasted_iota(jnp.int32, (128, 128), 1)).astype(jnp.float32)
    seg = jnp.dot(pc, upper, preferred_element_type=jnp.float32)

    pos = seg + rank
    inv0_ref[...] = jnp.sum(pos * oh0, axis=1, keepdims=True).astype(jnp.int32)
    inv1_ref[...] = jnp.sum(pos * oh1, axis=1, keepdims=True).astype(jnp.int32)

    bstart = (lax.broadcasted_iota(jnp.int32, (NB, _LANES), 0) * G
              ).astype(jnp.float32)
    lane_b = lax.broadcasted_iota(jnp.int32, (NB, _LANES), 1)
    ind = (bstart >= seg) & (bstart < seg + pc)
    be_ref[...] = jnp.sum(
        jnp.where(ind, lane_b, 0), axis=1, keepdims=True).astype(jnp.int32)


def _route_plan(flat, wr_pad):
    return pl.pallas_call(
        _route_body,
        out_shape=(
            jax.ShapeDtypeStruct((BS, 1), jnp.int32),
            jax.ShapeDtypeStruct((BS, 1), jnp.int32),
            jax.ShapeDtypeStruct((BS, 1), jnp.float32),
            jax.ShapeDtypeStruct((BS, 1), jnp.float32),
            jax.ShapeDtypeStruct((NB, 1), jnp.int32),
        ),
        scratch_shapes=[pltpu.VMEM((BS, _LANES), jnp.float32)],
        compiler_params=pltpu.CompilerParams(
            vmem_limit_bytes=64 * 1024 * 1024),
    )(flat, wr_pad)


def _sc_scatter_rows(flat, idx):
    bpw = NA // NW

    @functools.partial(
        pl.kernel,
        mesh=plsc.VectorSubcoreMesh(core_axis_name="c", subcore_axis_name="s"),
        out_type=jax.ShapeDtypeStruct((PAD, D), jnp.float32),
        scratch_types=[
            pltpu.VMEM((SC_CH,), jnp.int32),
            pltpu.VMEM((SC_CH, D), jnp.float32),
            pltpu.SemaphoreType.DMA,
        ],
    )
    def k(x_hbm, idx_hbm, xs_hbm, idx_v, rows_v, sem):
        wid = lax.axis_index("s") * 2 + lax.axis_index("c")
        base = wid * bpw

        @pl.loop(0, bpw // SC_CH)
        def _(ci):
            off = base + ci * SC_CH
            xoff = lax.rem(off, BS)
            pltpu.sync_copy(idx_hbm.at[pl.ds(off, SC_CH)], idx_v)
            pltpu.sync_copy(x_hbm.at[pl.ds(xoff, SC_CH)], rows_v)
            pltpu.async_copy(rows_v, xs_hbm.at[idx_v], sem).wait()

    return k(flat, idx)


def _grouped_body(be_ref, xs_ref, wg_ref, wu_ref, wd_ref, ys_ref):
    xb = xs_ref[...].astype(jnp.bfloat16)
    g = jnp.dot(xb, wg_ref[0], preferred_element_type=jnp.float32)
    u = jnp.dot(xb, wu_ref[0], preferred_element_type=jnp.float32)
    h = (g * jax.nn.sigmoid(g) * u).astype(jnp.bfloat16)
    ys_ref[...] = jnp.dot(h, wd_ref[0], preferred_element_type=jnp.float32)


def _grouped_mlp(be, xs, wg, wu, wd):
    grid_spec = pltpu.PrefetchScalarGridSpec(
        num_scalar_prefetch=1,
        grid=(NB,),
        in_specs=[
            pl.BlockSpec((G, D), lambda i, be: (i, 0)),
            pl.BlockSpec((1, D, F), lambda i, be: (be[i], 0, 0)),
            pl.BlockSpec((1, D, F), lambda i, be: (be[i], 0, 0)),
            pl.BlockSpec((1, F, D), lambda i, be: (be[i], 0, 0)),
        ],
        out_specs=pl.BlockSpec((G, D), lambda i, be: (i, 0)),
    )
    return pl.pallas_call(
        _grouped_body,
        grid_spec=grid_spec,
        out_shape=jax.ShapeDtypeStruct((PAD, D), jnp.float32),
        compiler_params=pltpu.CompilerParams(
            dimension_semantics=("arbitrary",)),
    )(be, xs, wg, wu, wd)


def _sc_gather_rows(ys, idx):
    bpw = NA // NW

    @functools.partial(
        pl.kernel,
        mesh=plsc.VectorSubcoreMesh(core_axis_name="c", subcore_axis_name="s"),
        out_type=jax.ShapeDtypeStruct((NA, D), jnp.float32),
        scratch_types=[
            pltpu.VMEM((SC_CH,), jnp.int32),
            pltpu.VMEM((SC_CH, D), jnp.float32),
            pltpu.SemaphoreType.DMA,
        ],
    )
    def k(ys_hbm, idx_hbm, g_hbm, idx_v, rows_v, sem):
        wid = lax.axis_index("s") * 2 + lax.axis_index("c")
        base = wid * bpw

        @pl.loop(0, bpw // SC_CH)
        def _(ci):
            off = base + ci * SC_CH
            pltpu.sync_copy(idx_hbm.at[pl.ds(off, SC_CH)], idx_v)
            pltpu.async_copy(ys_hbm.at[idx_v], rows_v, sem).wait()
            pltpu.sync_copy(rows_v, g_hbm.at[pl.ds(off, SC_CH)])

    return k(ys, idx)


def _combine_body(g0_ref, g1_ref, w0_ref, w1_ref, out_ref):
    out_ref[...] = w0_ref[...] * g0_ref[...] + w1_ref[...] * g1_ref[...]


_RB = 512


def _combine(g, w0, w1):
    nblk = BS // _RB
    return pl.pallas_call(
        _combine_body,
        grid=(nblk,),
        in_specs=[
            pl.BlockSpec((_RB, D), lambda i: (i, 0)),
            pl.BlockSpec((_RB, D), lambda i: (i + nblk, 0)),
            pl.BlockSpec((_RB, 1), lambda i: (i, 0)),
            pl.BlockSpec((_RB, 1), lambda i: (i, 0)),
        ],
        out_specs=pl.BlockSpec((_RB, D), lambda i: (i, 0)),
        out_shape=jax.ShapeDtypeStruct((BS, D), jnp.float32),
    )(g, g, w0, w1)


def kernel(hidden_states, W_router, W_gate, W_up, W_down):
    B, S, H = hidden_states.shape
    flat = hidden_states.reshape(BS, D)
    wr_pad = jnp.pad(W_router, ((0, 0), (0, _LANES - NE)))

    inv0, inv1, w0, w1, be = _route_plan(flat, wr_pad)
    idx = jnp.concatenate([inv0.reshape(BS), inv1.reshape(BS)])

    xs = _sc_scatter_rows(flat, idx)
    ys = _grouped_mlp(be.reshape(NB),
                      xs,
                      W_gate.astype(jnp.bfloat16),
                      W_up.astype(jnp.bfloat16),
                      W_down.astype(jnp.bfloat16))
    g = _sc_gather_rows(ys, idx)
    out = _combine(g, w0, w1)
    return out.reshape(B, S, H)


# speedup vs baseline: 1.1105x; 1.1105x over previous
import functools

import jax
import jax.numpy as jnp
from jax import lax
from jax.experimental import pallas as pl
from jax.experimental.pallas import tpu as pltpu
from jax.experimental.pallas import tpu_sc as plsc

NE = 8
D = 1024
F = 1024
BS = 4096
NA = 2 * BS
G = 256
NB = (NA + NE * (G - 1) + G - 1) // G
PAD = NB * G

_LANES = 128

NW = 32
SC_CH = 64


def _route_body(x_ref, wr_ref, inv0_ref, inv1_ref, w0_ref, w1_ref, be_ref,
                rank_ref):
    x = x_ref[...]
    wr = wr_ref[...]
    logits = jnp.dot(x, wr, preferred_element_type=jnp.float32)
    lane = lax.broadcasted_iota(jnp.int32, (BS, _LANES), 1)
    valid = lane < NE
    neg = jnp.float32(-1e30)
    ml = jnp.where(valid, logits, neg)

    m1 = jnp.max(ml, axis=1, keepdims=True)
    idx1 = jnp.min(jnp.where(ml == m1, lane, _LANES), axis=1, keepdims=True)
    oh0 = (lane == idx1).astype(jnp.float32)
    ml2 = jnp.where(lane == idx1, neg, ml)
    m2 = jnp.max(ml2, axis=1, keepdims=True)
    idx2 = jnp.min(jnp.where(ml2 == m2, lane, _LANES), axis=1, keepdims=True)
    oh1 = (lane == idx2).astype(jnp.float32)

    w0_ref[...] = jax.nn.sigmoid(m1 - m2)
    w1_ref[...] = jax.nn.sigmoid(m2 - m1)

    rank_ref[...] = oh0 + oh1
    tri = (lax.broadcasted_iota(jnp.int32, (128, 128), 0) >
           lax.broadcasted_iota(jnp.int32, (128, 128), 1)).astype(jnp.float32)

    def chunk(c, carry):
        ch = rank_ref[pl.ds(c * 128, 128), :]
        within = jnp.dot(tri, ch, preferred_element_type=jnp.float32)
        rank_ref[pl.ds(c * 128, 128), :] = within + carry
        return carry + jnp.sum(ch, axis=0, keepdims=True)

    counts = lax.fori_loop(0, BS // 128, chunk,
                           jnp.zeros((1, _LANES), jnp.float32))
    rank = rank_ref[...]

    pc = jnp.floor((counts + (G - 1)) * (1.0 / G)) * G
    upper = (lax.broadcasted_iota(jnp.int32, (128, 128), 0) <
             lax.broadcasted_iota(jnp.int32, (128, 128), 1)).astype(jnp.float32)
    seg = jnp.dot(pc, upper, preferred_element_type=jnp.float32)

    pos = seg + rank
    inv0_ref[...] = jnp.sum(pos * oh0, axis=1, keepdims=True).astype(jnp.int32)
    inv1_ref[...] = jnp.sum(pos * oh1, axis=1, keepdims=True).astype(jnp.int32)

    bstart = (lax.broadcasted_iota(jnp.int32, (NB, _LANES), 0) * G
              ).astype(jnp.float32)
    lane_b = lax.broadcasted_iota(jnp.int32, (NB, _LANES), 1)
    ind = (bstart >= seg) & (bstart < seg + pc)
    be_ref[...] = jnp.sum(
        jnp.where(ind, lane_b, 0), axis=1, keepdims=True).astype(jnp.int32)


def _route_plan(flat, wr_pad):
    return pl.pallas_call(
        _route_body,
        out_shape=(
            jax.ShapeDtypeStruct((BS, 1), jnp.int32),
            jax.ShapeDtypeStruct((BS, 1), jnp.int32),
            jax.ShapeDtypeStruct((BS, 1), jnp.float32),
            jax.ShapeDtypeStruct((BS, 1), jnp.float32),
            jax.ShapeDtypeStruct((NB, 1), jnp.int32),
        ),
        scratch_shapes=[pltpu.VMEM((BS, _LANES), jnp.float32)],
        compiler_params=pltpu.CompilerParams(
            vmem_limit_bytes=64 * 1024 * 1024),
    )(flat, wr_pad)


def _sc_scatter_rows(flat, idx):
    bpw = NA // NW

    @functools.partial(
        pl.kernel,
        mesh=plsc.VectorSubcoreMesh(core_axis_name="c", subcore_axis_name="s"),
        out_type=jax.ShapeDtypeStruct((PAD, D), jnp.float32),
        scratch_types=[
            pltpu.VMEM((SC_CH,), jnp.int32),
            pltpu.VMEM((SC_CH, D), jnp.float32),
            pltpu.SemaphoreType.DMA,
        ],
    )
    def k(x_hbm, idx_hbm, xs_hbm, idx_v, rows_v, sem):
        wid = lax.axis_index("s") * 2 + lax.axis_index("c")
        base = wid * bpw

        @pl.loop(0, bpw // SC_CH)
        def _(ci):
            off = base + ci * SC_CH
            xoff = lax.rem(off, BS)
            pltpu.sync_copy(idx_hbm.at[pl.ds(off, SC_CH)], idx_v)
            pltpu.sync_copy(x_hbm.at[pl.ds(xoff, SC_CH)], rows_v)
            pltpu.async_copy(rows_v, xs_hbm.at[idx_v], sem).wait()

    return k(flat, idx)


def _grouped_body(be_ref, xs_ref, wg_ref, wu_ref, wd_ref, ys_ref):
    xb = xs_ref[...]
    g = jnp.dot(xb, wg_ref[0], preferred_element_type=jnp.float32)
    u = jnp.dot(xb, wu_ref[0], preferred_element_type=jnp.float32)
    h = g * jax.nn.sigmoid(g) * u
    ys_ref[...] = jnp.dot(h, wd_ref[0], preferred_element_type=jnp.float32)


def _grouped_mlp(be, xs, wg, wu, wd):
    grid_spec = pltpu.PrefetchScalarGridSpec(
        num_scalar_prefetch=1,
        grid=(NB,),
        in_specs=[
            pl.BlockSpec((G, D), lambda i, be: (i, 0)),
            pl.BlockSpec((1, D, F), lambda i, be: (be[i], 0, 0)),
            pl.BlockSpec((1, D, F), lambda i, be: (be[i], 0, 0)),
            pl.BlockSpec((1, F, D), lambda i, be: (be[i], 0, 0)),
        ],
        out_specs=pl.BlockSpec((G, D), lambda i, be: (i, 0)),
    )
    return pl.pallas_call(
        _grouped_body,
        grid_spec=grid_spec,
        out_shape=jax.ShapeDtypeStruct((PAD, D), jnp.float32),
        compiler_params=pltpu.CompilerParams(
            dimension_semantics=("arbitrary",)),
    )(be, xs, wg, wu, wd)


def _sc_gather_rows(ys, idx):
    bpw = NA // NW

    @functools.partial(
        pl.kernel,
        mesh=plsc.VectorSubcoreMesh(core_axis_name="c", subcore_axis_name="s"),
        out_type=jax.ShapeDtypeStruct((NA, D), jnp.float32),
        scratch_types=[
            pltpu.VMEM((SC_CH,), jnp.int32),
            pltpu.VMEM((SC_CH, D), jnp.float32),
            pltpu.SemaphoreType.DMA,
        ],
    )
    def k(ys_hbm, idx_hbm, g_hbm, idx_v, rows_v, sem):
        wid = lax.axis_index("s") * 2 + lax.axis_index("c")
        base = wid * bpw

        @pl.loop(0, bpw // SC_CH)
        def _(ci):
            off = base + ci * SC_CH
            pltpu.sync_copy(idx_hbm.at[pl.ds(off, SC_CH)], idx_v)
            pltpu.async_copy(ys_hbm.at[idx_v], rows_v, sem).wait()
            pltpu.sync_copy(rows_v, g_hbm.at[pl.ds(off, SC_CH)])

    return k(ys, idx)


def _combine_body(g0_ref, g1_ref, w0_ref, w1_ref, out_ref):
    out_ref[...] = w0_ref[...] * g0_ref[...] + w1_ref[...] * g1_ref[...]


_RB = 512


def _combine(g, w0, w1):
    nblk = BS // _RB
    return pl.pallas_call(
        _combine_body,
        grid=(nblk,),
        in_specs=[
            pl.BlockSpec((_RB, D), lambda i: (i, 0)),
            pl.BlockSpec((_RB, D), lambda i: (i + nblk, 0)),
            pl.BlockSpec((_RB, 1), lambda i: (i, 0)),
            pl.BlockSpec((_RB, 1), lambda i: (i, 0)),
        ],
        out_specs=pl.BlockSpec((_RB, D), lambda i: (i, 0)),
        out_shape=jax.ShapeDtypeStruct((BS, D), jnp.float32),
    )(g, g, w0, w1)


def kernel(hidden_states, W_router, W_gate, W_up, W_down):
    B, S, H = hidden_states.shape
    flat = hidden_states.reshape(BS, D)
    wr_pad = jnp.pad(W_router, ((0, 0), (0, _LANES - NE)))

    inv0, inv1, w0, w1, be = _route_plan(flat, wr_pad)
    idx = jnp.concatenate([inv0.reshape(BS), inv1.reshape(BS)])

    xs = _sc_scatter_rows(flat, idx)
    ys = _grouped_mlp(be.reshape(NB), xs, W_gate, W_up, W_down)
    g = _sc_gather_rows(ys, idx)
    out = _combine(g, w0, w1)
    return out.reshape(B, S, H)
